# Initial kernel scaffold; baseline (speedup 1.0000x reference)
import jax
import jax.numpy as jnp
from jax.experimental import pallas as pl


def kernel(batch, table):
    raise NotImplementedError("write your pallas kernel here")


# speedup vs baseline: 7.7589x; 7.7589x over previous
import functools

import jax
import jax.numpy as jnp
from jax import lax
from jax.experimental import pallas as pl
from jax.experimental.pallas import tpu as pltpu
from jax.experimental.pallas import tpu_sc as plsc

_INFO = plsc.get_sparse_core_info()
_NC = _INFO.num_cores
_NS = _INFO.num_subcores
_NW = _NC * _NS

_CHUNK = 800


def _gather_body(n_chunks, table_hbm, idx_hbm, out_hbm, idx_v, rows_v, sem):
    wid = lax.axis_index("s") * _NC + lax.axis_index("c")
    base = wid * (n_chunks * _CHUNK)

    def step(i, carry):
        off = pl.multiple_of(base + i * _CHUNK, 8)
        pltpu.sync_copy(idx_hbm.at[pl.ds(off, _CHUNK)], idx_v)
        pltpu.async_copy(table_hbm.at[idx_v], rows_v, sem).wait()
        pltpu.sync_copy(rows_v, out_hbm.at[pl.ds(off, _CHUNK)])
        return carry

    lax.fori_loop(0, n_chunks, step, 0)


@functools.partial(jax.jit, static_argnames=("b", "l", "d"))
def _lookup(batch_flat, table, b, l, d):
    n = b * l
    assert n % (_NW * _CHUNK) == 0
    n_chunks = n // (_NW * _CHUNK)
    mesh = plsc.VectorSubcoreMesh(core_axis_name="c", subcore_axis_name="s")
    out = pl.kernel(
        functools.partial(_gather_body, n_chunks),
        out_type=jax.ShapeDtypeStruct((n, d), jnp.float32),
        mesh=mesh,
        scratch_types=[
            pltpu.VMEM((_CHUNK,), jnp.int32),
            pltpu.VMEM((_CHUNK, d), jnp.float32),
            pltpu.SemaphoreType.DMA,
        ],
    )(table, batch_flat)
    return out.reshape(b, l, d)


def kernel(batch, table):
    b, l = batch.shape
    d = table.shape[1]
    return _lookup(batch.reshape(-1).astype(jnp.int32), table, b, l, d)


# speedup vs baseline: 8.0493x; 1.0374x over previous
import functools

import jax
import jax.numpy as jnp
from jax import lax
from jax.experimental import pallas as pl
from jax.experimental.pallas import tpu as pltpu
from jax.experimental.pallas import tpu_sc as plsc

_INFO = plsc.get_sparse_core_info()
_NC = _INFO.num_cores
_NS = _INFO.num_subcores
_NW = _NC * _NS

_CHUNK = 400
_NBUF = 2


def _gather_body(n_chunks, table_hbm, idx_hbm, out_hbm,
                 i0, i1, r0, r1, g0, g1, w0, w1):
    idx_bufs = (i0, i1)
    row_bufs = (r0, r1)
    gsems = (g0, g1)
    wsems = (w0, w1)

    wid = lax.axis_index("s") * _NC + lax.axis_index("c")
    base = wid * (n_chunks * _CHUNK)

    def off(i):
        return pl.multiple_of(base + i * _CHUNK, 8)

    def start_gather(b):
        return pltpu.async_copy(table_hbm.at[idx_bufs[b]], row_bufs[b],
                                gsems[b])

    def wait_gather(b):
        pltpu.make_async_copy(table_hbm.at[idx_bufs[b]], row_bufs[b],
                              gsems[b]).wait()

    def start_write(b, i):
        return pltpu.async_copy(row_bufs[b], out_hbm.at[pl.ds(off(i), _CHUNK)],
                                wsems[b])

    for b in range(_NBUF):
        pltpu.sync_copy(idx_hbm.at[pl.ds(off(b), _CHUNK)], idx_bufs[b])
        start_gather(b)

    def step(g, carry):
        for b in range(_NBUF):
            i = _NBUF * g + b
            wait_gather(b)
            wr = start_write(b, i)
            pltpu.sync_copy(idx_hbm.at[pl.ds(off(i + _NBUF), _CHUNK)],
                            idx_bufs[b])
            wr.wait()
            start_gather(b)
        return carry

    lax.fori_loop(0, n_chunks // _NBUF - 1, step, 0)

    tail = n_chunks - _NBUF
    handles = []
    for b in range(_NBUF):
        wait_gather(b)
        handles.append(start_write(b, tail + b))
    for h in handles:
        h.wait()


@functools.partial(jax.jit, static_argnames=("b", "l", "d"))
def _lookup(batch_flat, table, b, l, d):
    n = b * l
    assert n % (_NW * _CHUNK) == 0
    n_chunks = n // (_NW * _CHUNK)
    assert n_chunks % _NBUF == 0 and n_chunks >= 2 * _NBUF
    mesh = plsc.VectorSubcoreMesh(core_axis_name="c", subcore_axis_name="s")
    out = pl.kernel(
        functools.partial(_gather_body, n_chunks),
        out_type=jax.ShapeDtypeStruct((n, d), jnp.float32),
        mesh=mesh,
        scratch_types=[
            pltpu.VMEM((_CHUNK,), jnp.int32),
            pltpu.VMEM((_CHUNK,), jnp.int32),
            pltpu.VMEM((_CHUNK, d), jnp.float32),
            pltpu.VMEM((_CHUNK, d), jnp.float32),
            pltpu.SemaphoreType.DMA,
            pltpu.SemaphoreType.DMA,
            pltpu.SemaphoreType.DMA,
            pltpu.SemaphoreType.DMA,
        ],
    )(table, batch_flat)
    return out.reshape(b, l, d)


def kernel(batch, table):
    b, l = batch.shape
    d = table.shape[1]
    return _lookup(batch.reshape(-1).astype(jnp.int32), table, b, l, d)
